# baseline (speedup 1.0000x reference)
import jax
import jax.numpy as jnp
from jax import lax
from jax.experimental import pallas as pl
from jax.experimental.pallas import tpu as pltpu
from jax.experimental.pallas import tpu_sc as plsc

N = 10000
E = 320000
D = 128
EPS = 0.0

NC = 2
NS = 16
NW = NC * NS
K = 80
G = 5
NCHUNK = 125
NGRP = NCHUNK // G
EPW = NCHUNK * K
NBUF = 4
GA = 3
SOFF = NBUF - GA - 1
ZB = 80
NB = N // ZB
BPT = -(-NB // NS)


def _sc_agg_kernel(x_hbm, edge_hbm, out_hbm,
                   acc, sidx, didx, rows, gsem, ssem, isem):
    cid = lax.axis_index("c")
    tid = lax.axis_index("s")
    wid = cid * NS + tid

    def istart(g):
        gb = g % 2
        pltpu.async_copy(edge_hbm.at[0, wid, g], sidx.at[gb], isem)
        pltpu.async_copy(edge_hbm.at[1, wid, g], didx.at[gb], isem)

    def iwait():
        pltpu.make_async_copy(edge_hbm.at[0, wid, 0], sidx.at[0], isem).wait()
        pltpu.make_async_copy(edge_hbm.at[1, wid, 0], didx.at[0], isem).wait()

    def gstart(c, b):
        gb = (c // G) % 2
        s = c % G
        pltpu.async_copy(x_hbm.at[sidx.at[gb, s]], rows.at[b], gsem.at[b])

    def gwait(b):
        pltpu.make_async_copy(x_hbm.at[pl.ds(0, K)], rows.at[b],
                              gsem.at[b]).wait()

    def sstart(c, b):
        gb = (c // G) % 2
        s = c % G
        pltpu.async_copy(rows.at[b], acc.at[didx.at[gb, s]], ssem.at[b],
                         add=True)

    def swait(b):
        pltpu.make_async_copy(x_hbm.at[pl.ds(0, K)], rows.at[b],
                              ssem.at[b]).wait()

    istart(0)

    zbuf = rows.at[NBUF - 1]

    def zero_body(i, _):
        r = i // (D // 16)
        c = (i % (D // 16)) * 16
        zbuf[r, pl.ds(c, 16)] = jnp.zeros((16,), jnp.float32)
        return 0

    lax.fori_loop(0, ZB * (D // 16), zero_body, 0)

    iwait()
    for c in range(GA):
        gstart(c, c)

    def zcopy_body(j, _):
        b = j * NS + tid

        @pl.when(b < NB)
        def _():
            pltpu.sync_copy(zbuf, acc.at[pl.ds(b * ZB, ZB)])

        return 0

    lax.fori_loop(0, BPT, zcopy_body, 0)
    plsc.subcore_barrier()

    def edge_body(i, _):
        b = i % NBUF
        g = i // G
        s = i % G

        gwait(b)
        pl.when(i >= NBUF - GA)(lambda: swait((i + GA) % NBUF))
        pl.when((s == SOFF) & (g + 1 < NGRP))(lambda: istart(g + 1))
        pl.when((s == G - GA) & (g + 1 < NGRP))(iwait)
        pl.when(i + GA < NCHUNK)(lambda: gstart(i + GA, (i + GA) % NBUF))
        sstart(i, b)
        return 0

    lax.fori_loop(0, NCHUNK, edge_body, 0)
    for t in range(NBUF - GA):
        swait((NCHUNK - (NBUF - GA) + t) % NBUF)
    plsc.subcore_barrier()

    def out_body(j, _):
        b = j * NS + tid

        @pl.when(b < NB)
        def _():
            ob = j % 2
            r = b * ZB

            @pl.when(j >= 2)
            def _():
                pltpu.make_async_copy(acc.at[pl.ds(0, ZB)],
                                      out_hbm.at[0, pl.ds(0, ZB)],
                                      ssem.at[ob]).wait()

            pltpu.async_copy(acc.at[pl.ds(r, ZB)],
                             out_hbm.at[cid, pl.ds(r, ZB)], ssem.at[ob])

        return 0

    lax.fori_loop(0, BPT, out_body, 0)
    for ob in range(2):
        pltpu.make_async_copy(acc.at[pl.ds(0, ZB)],
                              out_hbm.at[0, pl.ds(0, ZB)],
                              ssem.at[ob]).wait()


def _sc_aggregate(x, edges):
    mesh = plsc.VectorSubcoreMesh(
        core_axis_name="c", subcore_axis_name="s",
        num_cores=NC, num_subcores=NS)
    return pl.kernel(
        _sc_agg_kernel,
        out_type=jax.ShapeDtypeStruct((NC, N, D), jnp.float32),
        mesh=mesh,
        scratch_types=[
            pltpu.VMEM_SHARED((N, D), jnp.float32),
            pltpu.VMEM((2, G, K), jnp.int32),
            pltpu.VMEM((2, G, K), jnp.int32),
            pltpu.VMEM((NBUF, K, D), jnp.float32),
            pltpu.SemaphoreType.DMA((NBUF,)),
            pltpu.SemaphoreType.DMA((NBUF,)),
            pltpu.SemaphoreType.DMA,
        ],
    )(x, edges)


def _tc_mlp_kernel(x_ref, p_ref, w1_ref, b1_ref, w2_ref, b2_ref,
                   out_ref):
    s = (1.0 + EPS) * x_ref[...] + p_ref[0] + p_ref[1]
    h = jnp.dot(s, w1_ref[...], preferred_element_type=jnp.float32)
    h = jnp.maximum(h + b1_ref[...], 0.0)
    z = jnp.dot(h, w2_ref[...], preferred_element_type=jnp.float32)
    out_ref[...] = z + b2_ref[...]


def _tc_mlp(x, partials, W1, b1, W2, b2):
    BT = 2000
    grid = (N // BT,)
    row_spec = pl.BlockSpec((BT, D), lambda i: (i, 0))
    p_spec = pl.BlockSpec((NC, BT, D), lambda i: (0, i, 0))
    full = pl.BlockSpec((D, D), lambda i: (0, 0))
    bias = pl.BlockSpec((1, D), lambda i: (0, 0))
    return pl.pallas_call(
        _tc_mlp_kernel,
        grid=grid,
        in_specs=[row_spec, p_spec, full, bias, full, bias],
        out_specs=row_spec,
        out_shape=jax.ShapeDtypeStruct((N, D), jnp.float32),
    )(x, partials, W1, b1.reshape(1, D), W2, b2.reshape(1, D))


@jax.jit
def kernel(x, edge_index, W1, b1, W2, b2):
    edges = edge_index.reshape(2, NW, NGRP, G, K)
    partials = _sc_aggregate(x, edges)
    return _tc_mlp(x, partials, W1, b1, W2, b2)


# speedup vs baseline: 1.0095x; 1.0095x over previous
import jax
import jax.numpy as jnp
from jax import lax
from jax.experimental import pallas as pl
from jax.experimental.pallas import tpu as pltpu
from jax.experimental.pallas import tpu_sc as plsc

N = 10000
E = 320000
D = 128
EPS = 0.0

NC = 2
NS = 16
NW = NC * NS
K = 80
G = 5
NCHUNK = 125
NGRP = NCHUNK // G
EPW = NCHUNK * K
NBUF = 4
GA = 3
SOFF = NBUF - GA - 1
ZB = 80
NB = N // ZB
BPT = -(-NB // NS)


def _sc_agg_kernel(x_hbm, edge_hbm, out_hbm,
                   acc, sidx, didx, rows, gsem, ssem, isem):
    cid = lax.axis_index("c")
    tid = lax.axis_index("s")
    wid = cid * NS + tid

    def istart(g):
        gb = g % 2
        pltpu.async_copy(edge_hbm.at[0, wid, g], sidx.at[gb], isem)
        pltpu.async_copy(edge_hbm.at[1, wid, g], didx.at[gb], isem)

    def iwait():
        pltpu.make_async_copy(edge_hbm.at[0, wid, 0], sidx.at[0], isem).wait()
        pltpu.make_async_copy(edge_hbm.at[1, wid, 0], didx.at[0], isem).wait()

    def gstart(c, b):
        gb = (c // G) % 2
        s = c % G
        pltpu.async_copy(x_hbm.at[sidx.at[gb, s]], rows.at[b], gsem.at[b])

    def gwait(b):
        pltpu.make_async_copy(x_hbm.at[pl.ds(0, K)], rows.at[b],
                              gsem.at[b]).wait()

    def sstart(c, b):
        gb = (c // G) % 2
        s = c % G
        pltpu.async_copy(rows.at[b], acc.at[didx.at[gb, s]], ssem.at[b],
                         add=True)

    def swait(b):
        pltpu.make_async_copy(x_hbm.at[pl.ds(0, K)], rows.at[b],
                              ssem.at[b]).wait()

    istart(0)

    zbuf = rows.at[NBUF - 1]

    def zero_body(i, _):
        r = i // (D // 16)
        c = (i % (D // 16)) * 16
        zbuf[r, pl.ds(c, 16)] = jnp.zeros((16,), jnp.float32)
        return 0

    lax.fori_loop(0, ZB * (D // 16), zero_body, 0)

    iwait()
    for c in range(GA):
        gstart(c, c)

    def zcopy_body(j, _):
        b = j * NS + tid

        @pl.when(b < NB)
        def _():
            pltpu.sync_copy(zbuf, acc.at[pl.ds(b * ZB, ZB)])

        return 0

    lax.fori_loop(0, BPT, zcopy_body, 0)
    plsc.subcore_barrier()

    def edge_body(i, _):
        b = i % NBUF
        g = i // G
        s = i % G

        gwait(b)
        pl.when(i >= NBUF - GA)(lambda: swait((i + GA) % NBUF))
        pl.when((s == SOFF) & (g + 1 < NGRP))(lambda: istart(g + 1))
        pl.when((s == G - GA) & (g + 1 < NGRP))(iwait)
        pl.when(i + GA < NCHUNK)(lambda: gstart(i + GA, (i + GA) % NBUF))
        sstart(i, b)
        return 0

    lax.fori_loop(0, NCHUNK, edge_body, 0)
    for t in range(NBUF - GA):
        swait((NCHUNK - (NBUF - GA) + t) % NBUF)
    plsc.subcore_barrier()

    def out_body(j, _):
        b = j * NS + tid

        @pl.when(b < NB)
        def _():
            ob = j % 2
            r = b * ZB

            @pl.when(j >= 2)
            def _():
                pltpu.make_async_copy(rows.at[ob], out_hbm.at[0, pl.ds(0, ZB)],
                                      ssem.at[ob]).wait()

            pltpu.async_copy(acc.at[pl.ds(r, ZB)], rows.at[ob],
                             gsem.at[ob]).wait()
            pltpu.async_copy(rows.at[ob], out_hbm.at[cid, pl.ds(r, ZB)],
                             ssem.at[ob])

        return 0

    lax.fori_loop(0, BPT, out_body, 0)
    for ob in range(2):
        pltpu.make_async_copy(rows.at[ob], out_hbm.at[0, pl.ds(0, ZB)],
                              ssem.at[ob]).wait()


def _sc_aggregate(x, edges):
    mesh = plsc.VectorSubcoreMesh(
        core_axis_name="c", subcore_axis_name="s",
        num_cores=NC, num_subcores=NS)
    return pl.kernel(
        _sc_agg_kernel,
        out_type=jax.ShapeDtypeStruct((NC, N, D), jnp.float32),
        mesh=mesh,
        scratch_types=[
            pltpu.VMEM_SHARED((N, D), jnp.float32),
            pltpu.VMEM((2, G, K), jnp.int32),
            pltpu.VMEM((2, G, K), jnp.int32),
            pltpu.VMEM((NBUF, K, D), jnp.float32),
            pltpu.SemaphoreType.DMA((NBUF,)),
            pltpu.SemaphoreType.DMA((NBUF,)),
            pltpu.SemaphoreType.DMA,
        ],
    )(x, edges)


def _tc_mlp_kernel(x_ref, p_ref, w1_ref, b1_ref, w2_ref, b2_ref,
                   out_ref):
    s = (1.0 + EPS) * x_ref[...] + p_ref[0] + p_ref[1]
    h = jnp.dot(s, w1_ref[...], preferred_element_type=jnp.float32)
    h = jnp.maximum(h + b1_ref[...], 0.0)
    z = jnp.dot(h, w2_ref[...], preferred_element_type=jnp.float32)
    out_ref[...] = z + b2_ref[...]


def _tc_mlp(x, partials, W1, b1, W2, b2):
    BT = 2000
    grid = (N // BT,)
    row_spec = pl.BlockSpec((BT, D), lambda i: (i, 0))
    p_spec = pl.BlockSpec((NC, BT, D), lambda i: (0, i, 0))
    full = pl.BlockSpec((D, D), lambda i: (0, 0))
    bias = pl.BlockSpec((1, D), lambda i: (0, 0))
    return pl.pallas_call(
        _tc_mlp_kernel,
        grid=grid,
        in_specs=[row_spec, p_spec, full, bias, full, bias],
        out_specs=row_spec,
        out_shape=jax.ShapeDtypeStruct((N, D), jnp.float32),
    )(x, partials, W1, b1.reshape(1, D), W2, b2.reshape(1, D))


@jax.jit
def kernel(x, edge_index, W1, b1, W2, b2):
    edges = edge_index.reshape(2, NW, NGRP, G, K)
    partials = _sc_aggregate(x, edges)
    return _tc_mlp(x, partials, W1, b1, W2, b2)
